# Initial kernel scaffold; baseline (speedup 1.0000x reference)
import jax
import jax.numpy as jnp
from jax.experimental import pallas as pl


def kernel(src, dst, neg_dst, n_id, edge_index, e_id, t, msg, memory, w_t, b_t, Wq, Wk, Wv, We, Wskip, W1, W2, b1, Wf, bf):
    raise NotImplementedError("write your pallas kernel here")


# speedup vs baseline: 8.0553x; 8.0553x over previous
import functools

import jax
import jax.numpy as jnp
from jax import lax
from jax.experimental import pallas as pl
from jax.experimental.pallas import tpu as pltpu
from jax.experimental.pallas import tpu_sc as plsc

NUM_NODES = 100000
N_SUB = 20000
N_EDGES = 200000
N_EVENTS = 200000
BATCH = 8192
MEM_DIM = 172
MEM_PAD = 256
TIME_DIM = 100
HID = 128
HEADS = 2
DH = 64
AGG_W = 128

NC, NS = 2, 16
NW = NC * NS

E_PAD = 200704
N_SUB_PAD = 20480

f32 = jnp.float32


def _sc_mesh():
    return plsc.VectorSubcoreMesh(core_axis_name="c", subcore_axis_name="s",
                                  num_cores=NC, num_subcores=NS)


_RB1 = 2000


def _event_proj_body(t_ref, wt_ref, bt_ref, msg_ref, wet_ref, wem_ref, out_ref):
    te = jnp.cos(t_ref[...] * wt_ref[...] + bt_ref[...])
    out_ref[...] = (
        jnp.dot(te, wet_ref[...], preferred_element_type=f32)
        + jnp.dot(msg_ref[...], wem_ref[...], preferred_element_type=f32))


def _event_proj(t2, wt2, bt2, msg, wet, wem):
    grid = (N_EVENTS // _RB1,)
    return pl.pallas_call(
        _event_proj_body,
        grid=grid,
        in_specs=[
            pl.BlockSpec((_RB1, 1), lambda i: (i, 0)),
            pl.BlockSpec((1, TIME_DIM), lambda i: (0, 0)),
            pl.BlockSpec((1, TIME_DIM), lambda i: (0, 0)),
            pl.BlockSpec((_RB1, MEM_DIM), lambda i: (i, 0)),
            pl.BlockSpec((TIME_DIM, HID), lambda i: (0, 0)),
            pl.BlockSpec((MEM_DIM, HID), lambda i: (0, 0)),
        ],
        out_specs=pl.BlockSpec((_RB1, HID), lambda i: (i, 0)),
        out_shape=jax.ShapeDtypeStruct((N_EVENTS, HID), f32),
    )(t2, wt2, bt2, msg, wet, wem)


_RB2 = 2048


def _node_proj_body(mem_ref, wq_ref, wk_ref, wv_ref, ws_ref,
                    kv_ref, q_ref, s_ref):
    m = mem_ref[...]
    kv_ref[:, :HID] = jnp.dot(m, wk_ref[...], preferred_element_type=f32)
    kv_ref[:, HID:] = jnp.dot(m, wv_ref[...], preferred_element_type=f32)
    q_ref[...] = jnp.dot(m, wq_ref[...], preferred_element_type=f32)
    s_ref[...] = jnp.dot(m, ws_ref[...], preferred_element_type=f32)


def _node_proj(memg, wq, wk, wv, ws):
    grid = (N_SUB_PAD // _RB2,)
    wspec = pl.BlockSpec((MEM_PAD, HID), lambda i: (0, 0))
    return pl.pallas_call(
        _node_proj_body,
        grid=grid,
        in_specs=[pl.BlockSpec((_RB2, MEM_PAD), lambda i: (i, 0)),
                  wspec, wspec, wspec, wspec],
        out_specs=(pl.BlockSpec((_RB2, 2 * HID), lambda i: (i, 0)),
                   pl.BlockSpec((_RB2, HID), lambda i: (i, 0)),
                   pl.BlockSpec((_RB2, HID), lambda i: (i, 0))),
        out_shape=(jax.ShapeDtypeStruct((N_SUB_PAD, 2 * HID), f32),
                   jax.ShapeDtypeStruct((N_SUB_PAD, HID), f32),
                   jax.ShapeDtypeStruct((N_SUB_PAD, HID), f32)),
    )(memg, wq, wk, wv, ws)


def _sc_gather(table, idx, d):
    n = idx.shape[0]
    chunks = n // 128
    idx2 = idx.reshape(1, n)

    @functools.partial(
        pl.kernel,
        mesh=_sc_mesh(),
        out_type=jax.ShapeDtypeStruct((n, d), f32),
    )
    def k(table_hbm, idx_hbm, out_hbm):
        def body(i_vmem, o_vmem):
            pltpu.sync_copy(table_hbm.at[i_vmem.at[0]], o_vmem)

        pltpu.emit_pipeline(
            body,
            grid=(chunks,),
            in_specs=[pl.BlockSpec((1, 128), lambda i: (0, i))],
            out_specs=[pl.BlockSpec((128, d), lambda i: (i, 0))],
            core_axis_name=("c", "s"),
            dimension_semantics=(pltpu.PARALLEL,),
        )(idx_hbm, out_hbm)

    return k(table, idx2)


_CH4 = 128


def _edge_gather_kv(kvt, src2):
    @functools.partial(
        pl.kernel,
        mesh=_sc_mesh(),
        out_type=jax.ShapeDtypeStruct((E_PAD, 2 * HID), f32),
    )
    def kern(kv_h, src_h, okv):
        def body(is_v, okv_v):
            pltpu.sync_copy(kv_h.at[is_v.at[0]], okv_v)

        pltpu.emit_pipeline(
            body,
            grid=(E_PAD // _CH4,),
            in_specs=[pl.BlockSpec((1, _CH4), lambda i: (0, i))],
            out_specs=[pl.BlockSpec((_CH4, 2 * HID), lambda i: (i, 0))],
            core_axis_name=("c", "s"),
            dimension_semantics=(pltpu.PARALLEL,),
        )(src_h, okv)

    return kern(kvt, src2)


def _edge_gather_qe(qt, evt, dst2, eid2):
    row_t = jax.ShapeDtypeStruct((E_PAD, HID), f32)

    @functools.partial(
        pl.kernel,
        mesh=_sc_mesh(),
        out_type=(row_t, row_t),
    )
    def kern(q_h, ev_h, dst_h, eid_h, oq, oe):
        def body(id_v, ie_v, oq_v, oe_v):
            pltpu.sync_copy(q_h.at[id_v.at[0]], oq_v)
            pltpu.sync_copy(ev_h.at[ie_v.at[0]], oe_v)

        ispec = pl.BlockSpec((1, _CH4), lambda i: (0, i))
        ospec = pl.BlockSpec((_CH4, HID), lambda i: (i, 0))
        pltpu.emit_pipeline(
            body,
            grid=(E_PAD // _CH4,),
            in_specs=[ispec, ispec],
            out_specs=[ospec, ospec],
            core_axis_name=("c", "s"),
            dimension_semantics=(pltpu.PARALLEL,),
        )(dst_h, eid_h, oq, oe)

    return kern(qt, evt, dst2, eid2)


_RB5 = 2048


def _edge_math_body(kv_ref, qd_ref, ee_ref, out_ref):
    i = pl.program_id(0)
    ee = ee_ref[...]
    ke = kv_ref[:, :HID] + ee
    ve = kv_ref[:, HID:] + ee
    prod = qd_ref[...] * ke
    a0 = jnp.sum(prod[:, :DH], axis=1, keepdims=True) * 0.125
    a1 = jnp.sum(prod[:, DH:], axis=1, keepdims=True) * 0.125
    rowid = lax.broadcasted_iota(jnp.int32, (_RB5, 1), 0) + i * _RB5
    valid = rowid < N_EDGES
    e0 = jnp.where(valid, jnp.exp(a0), 0.0)
    e1 = jnp.where(valid, jnp.exp(a1), 0.0)
    pad = jnp.zeros((_RB5, AGG_W - DH - 1), f32)
    h0 = jnp.concatenate([e0 * ve[:, :DH], e0, pad], axis=1)
    h1 = jnp.concatenate([e1 * ve[:, DH:], e1, pad], axis=1)
    out_ref[...] = jnp.stack([h0, h1], axis=0)


def _edge_math(kvse, qdst, ee):
    grid = (E_PAD // _RB5,)
    espec = pl.BlockSpec((_RB5, HID), lambda i: (i, 0))
    return pl.pallas_call(
        _edge_math_body,
        grid=grid,
        in_specs=[pl.BlockSpec((_RB5, 2 * HID), lambda i: (i, 0)), espec, espec],
        out_specs=pl.BlockSpec((HEADS, _RB5, AGG_W), lambda i: (0, i, 0)),
        out_shape=jax.ShapeDtypeStruct((HEADS, E_PAD, AGG_W), f32),
    )(kvse, qdst, ee)


_CH6 = 128
_CPT6 = E_PAD // (NS * _CH6)
_HALF_N = 10240
_TRASH = _HALF_N
_AGG_ROWS = _HALF_N + 8
N_AGG = 2 * _HALF_N
_ZR6 = _HALF_N // NS


def _seg_scatter(evw, dst3, zrows):
    @functools.partial(
        pl.kernel,
        mesh=_sc_mesh(),
        out_type=jax.ShapeDtypeStruct((HEADS, N_AGG, AGG_W), f32),
        scratch_types=[
            pltpu.VMEM((_CPT6, _CH6), jnp.int32),
            pltpu.VMEM((_CH6, AGG_W), f32),
            pltpu.VMEM((_CH6, AGG_W), f32),
            pltpu.VMEM_SHARED((_AGG_ROWS, AGG_W), f32),
        ],
    )
    def kern(evw_h, dst_h, z_h, out_h, idx_v, buf, zbuf, agg):
        c = lax.axis_index("c")
        s = lax.axis_index("s")
        pltpu.sync_copy(z_h, zbuf)

        for p in range(2):
            lo = p * _HALF_N
            pltpu.sync_copy(dst_h.at[s], idx_v)

            @pl.loop(0, _CPT6 * (_CH6 // 16))
            def _(t):
                j = t // (_CH6 // 16)
                k = t % (_CH6 // 16)
                v = idx_v[j, pl.ds(k * 16, 16)]
                m = (v >= lo) & (v < lo + _HALF_N)
                idx_v[j, pl.ds(k * 16, 16)] = jnp.where(m, v - lo, _TRASH)

            @pl.loop(0, _ZR6 // _CH6)
            def _(r):
                pltpu.sync_copy(zbuf, agg.at[pl.ds(s * _ZR6 + r * _CH6, _CH6)])

            @pl.when(s == 0)
            def _():
                pltpu.sync_copy(zbuf.at[pl.ds(0, 8)],
                                agg.at[pl.ds(_HALF_N, 8)])

            plsc.subcore_barrier()

            @pl.loop(0, _CPT6)
            def _(j):
                pltpu.sync_copy(
                    evw_h.at[c, pl.ds(s * _CPT6 * _CH6 + j * _CH6, _CH6)], buf)
                pltpu.sync_copy(buf, agg.at[idx_v.at[j]], add=True)

            plsc.subcore_barrier()

            @pl.loop(0, _ZR6 // _CH6)
            def _(r):
                pltpu.sync_copy(agg.at[pl.ds(s * _ZR6 + r * _CH6, _CH6)], zbuf)
                pltpu.sync_copy(
                    zbuf, out_h.at[c, pl.ds(lo + s * _ZR6 + r * _CH6, _CH6)])

            pltpu.sync_copy(z_h, zbuf)

    return kern(evw, dst3, zrows)


_RB7 = 2000


def _z_comb_body(agg_ref, skip_ref, z_ref):
    a0 = agg_ref[0]
    a1 = agg_ref[1]
    z0 = a0[:, :DH] / (a0[:, DH:DH + 1] + 1e-16)
    z1 = a1[:, :DH] / (a1[:, DH:DH + 1] + 1e-16)
    z_ref[...] = jnp.concatenate([z0, z1], axis=1) + skip_ref[...]


def _z_comb(agg, skip):
    grid = (N_SUB // _RB7,)
    return pl.pallas_call(
        _z_comb_body,
        grid=grid,
        in_specs=[pl.BlockSpec((HEADS, _RB7, AGG_W), lambda i: (0, i, 0)),
                  pl.BlockSpec((_RB7, HID), lambda i: (i, 0))],
        out_specs=pl.BlockSpec((_RB7, HID), lambda i: (i, 0)),
        out_shape=jax.ShapeDtypeStruct((N_SUB, HID), f32),
    )(agg, skip)


def _softplus(x):
    return jnp.maximum(x, 0.0) + jnp.log1p(jnp.exp(-jnp.abs(x)))


def _pred_body(z3_ref, w1_ref, w2_ref, b1_ref, wf_ref, bf_ref, out_ref):
    zs = z3_ref[:BATCH]
    zd = z3_ref[BATCH:2 * BATCH]
    zn = z3_ref[2 * BATCH:]
    zw1 = jnp.dot(zs, w1_ref[...], preferred_element_type=f32)
    w2 = w2_ref[...]
    b1 = b1_ref[...]
    wf = wf_ref[...]
    bf = bf_ref[0, 0]
    hp = jax.nn.relu(zw1 + jnp.dot(zd, w2, preferred_element_type=f32) + b1)
    hn = jax.nn.relu(zw1 + jnp.dot(zn, w2, preferred_element_type=f32) + b1)
    pos = jnp.dot(hp, wf, preferred_element_type=f32) + bf
    neg = jnp.dot(hn, wf, preferred_element_type=f32) + bf
    loss = jnp.mean(_softplus(-pos)) + jnp.mean(_softplus(neg))
    out_ref[...] = loss[None, None]


def _predictor(z3, w1, w2, b12, wf, bf2):
    return pl.pallas_call(
        _pred_body,
        out_shape=jax.ShapeDtypeStruct((1, 1), f32),
    )(z3, w1, w2, b12, wf, bf2)


def kernel(src, dst, neg_dst, n_id, edge_index, e_id, t, msg, memory,
           w_t, b_t, Wq, Wk, Wv, We, Wskip, W1, W2, b1, Wf, bf):
    i32 = jnp.int32
    src_l = edge_index[0].astype(i32)
    dst_l = edge_index[1].astype(i32)
    epad = E_PAD - N_EDGES

    ev = _event_proj(t.reshape(N_EVENTS, 1), w_t.reshape(1, TIME_DIM),
                     b_t.reshape(1, TIME_DIM), msg,
                     We[:TIME_DIM], We[TIME_DIM:])

    mem_pad = jnp.pad(memory, ((0, 0), (0, MEM_PAD - MEM_DIM)))
    nidp = jnp.pad(n_id.astype(i32), (0, N_SUB_PAD - N_SUB))
    memg = _sc_gather(mem_pad, nidp, MEM_PAD)

    def wpad(w):
        return jnp.pad(w, ((0, MEM_PAD - MEM_DIM), (0, 0)))
    kv, q, skip = _node_proj(memg, wpad(Wq), wpad(Wk), wpad(Wv), wpad(Wskip))

    src2 = jnp.pad(src_l, (0, epad)).reshape(1, E_PAD)
    dst2 = jnp.pad(dst_l, (0, epad)).reshape(1, E_PAD)
    eid2 = jnp.pad(e_id.astype(i32), (0, epad)).reshape(1, E_PAD)
    kvse = _edge_gather_kv(kv, src2)
    qdst, ee = _edge_gather_qe(q, ev, dst2, eid2)

    evw = _edge_math(kvse, qdst, ee)

    dstt = jnp.pad(dst_l, (0, epad)).reshape(NS, _CPT6, _CH6)
    agg = _seg_scatter(evw, dstt, jnp.zeros((_CH6, AGG_W), f32))

    z = _z_comb(agg, skip)

    assoc = jnp.zeros((NUM_NODES,), i32).at[n_id].set(
        jnp.arange(N_SUB, dtype=i32))
    loc = assoc[jnp.concatenate([src, dst, neg_dst])].astype(i32)
    z3 = _sc_gather(z, loc, HID)

    loss = _predictor(z3, W1, W2, b1.reshape(1, HID), Wf, bf.reshape(1, 1))
    return loss[0, 0]
